# baseline (speedup 1.0000x reference)
import functools

import jax
import jax.numpy as jnp
from jax import lax
from jax.experimental import pallas as pl
from jax.experimental.pallas import tpu as pltpu
from jax.experimental.pallas import tpu_sc as plsc

NC = 2
NS = 16
LANES = 16
CHUNK = 512


def _emb_body(per_w, n_chunks, D,
              ids_hbm, tids_hbm, tok_hbm, typ_hbm, out_hbm,
              idx_v, tidx_v, rows_v, trows_v, sem0, sem1):
    wid = lax.axis_index("s") * NC + lax.axis_index("c")
    w_base = wid * per_w

    def chunk_body(ci, carry):
        base = w_base + ci * CHUNK
        pltpu.sync_copy(ids_hbm.at[pl.ds(base, CHUNK)], idx_v)
        pltpu.sync_copy(tids_hbm.at[pl.ds(base, CHUNK)], tidx_v)
        cp0 = pltpu.async_copy(tok_hbm.at[idx_v], rows_v, sem0)
        cp1 = pltpu.async_copy(typ_hbm.at[tidx_v], trows_v, sem1)
        cp0.wait()
        cp1.wait()

        @plsc.parallel_loop(0, CHUNK, step=1, unroll=8)
        def add_body(t):
            for d in range(D // LANES):
                sl = pl.ds(d * LANES, LANES)
                rows_v[t, sl] = rows_v[t, sl] + trows_v[t, sl]
        pltpu.sync_copy(rows_v, out_hbm.at[pl.ds(base, CHUNK)])
        return carry

    lax.fori_loop(0, n_chunks, chunk_body, 0)


def kernel(input_ids, type_ids, token_table, type_table):
    B, S = input_ids.shape
    V, D = token_table.shape
    N = B * S
    NW = NC * NS
    per_w = N // NW
    n_chunks = per_w // CHUNK
    assert per_w * NW == N and n_chunks * CHUNK == per_w

    ids = input_ids.reshape(N).astype(jnp.int32)
    tids = type_ids.reshape(N).astype(jnp.int32)

    mesh = plsc.VectorSubcoreMesh(
        core_axis_name="c", subcore_axis_name="s",
        num_cores=NC, num_subcores=NS)

    emb = functools.partial(
        pl.kernel,
        out_type=jax.ShapeDtypeStruct((N, D), jnp.float32),
        mesh=mesh,
        scratch_types=[
            pltpu.VMEM((CHUNK,), jnp.int32),
            pltpu.VMEM((CHUNK,), jnp.int32),
            pltpu.VMEM((CHUNK, D), jnp.float32),
            pltpu.VMEM((CHUNK, D), jnp.float32),
            pltpu.SemaphoreType.DMA,
            pltpu.SemaphoreType.DMA,
        ],
        compiler_params=pltpu.CompilerParams(use_tc_tiling_on_sc=False),
    )(functools.partial(_emb_body, per_w, n_chunks, D))

    out = emb(ids, tids, token_table, type_table)
    return out.reshape(B, S, D)


# speedup vs baseline: 12.2851x; 12.2851x over previous
import functools

import jax
import jax.numpy as jnp
from jax import lax
from jax.experimental import pallas as pl
from jax.experimental.pallas import tpu as pltpu
from jax.experimental.pallas import tpu_sc as plsc

NC = 2
NS = 16
LANES = 16
CHUNK = 512


def _emb_body(per_w, n_chunks, D,
              ids_hbm, tids_hbm, tok_hbm, typ_hbm, out_hbm,
              idx_v, tidx_v, rows_v, typ_v, sem0):
    wid = lax.axis_index("s") * NC + lax.axis_index("c")
    w_base = wid * per_w
    nd = D // LANES

    pltpu.sync_copy(typ_hbm, typ_v)
    row0 = [typ_v[pl.ds(d * LANES, LANES)] for d in range(nd)]
    diff = [typ_v[pl.ds(D + d * LANES, LANES)] - row0[d] for d in range(nd)]

    def chunk_body(ci, carry):
        base = w_base + ci * CHUNK
        pltpu.sync_copy(ids_hbm.at[pl.ds(base, CHUNK)], idx_v)
        pltpu.sync_copy(tids_hbm.at[pl.ds(base, CHUNK)], tidx_v)
        pltpu.async_copy(tok_hbm.at[idx_v], rows_v, sem0).wait()

        @plsc.parallel_loop(0, CHUNK, step=LANES, unroll=2)
        def add_body(t0):
            tid16 = tidx_v[pl.ds(t0, LANES)].astype(jnp.float32)
            for l in range(LANES):
                tidf = tid16[l]
                for d in range(nd):
                    sl = pl.ds(d * LANES, LANES)
                    rows_v[t0 + l, sl] = rows_v[t0 + l, sl] + (
                        row0[d] + tidf * diff[d])

        pltpu.sync_copy(rows_v, out_hbm.at[pl.ds(base, CHUNK)])
        return carry

    lax.fori_loop(0, n_chunks, chunk_body, 0)


def kernel(input_ids, type_ids, token_table, type_table):
    B, S = input_ids.shape
    V, D = token_table.shape
    N = B * S
    NW = NC * NS
    per_w = N // NW
    n_chunks = per_w // CHUNK
    assert per_w * NW == N and n_chunks * CHUNK == per_w

    ids = input_ids.reshape(N).astype(jnp.int32)
    tids = type_ids.reshape(N).astype(jnp.int32)
    typ = type_table.reshape(2 * D)

    mesh = plsc.VectorSubcoreMesh(
        core_axis_name="c", subcore_axis_name="s",
        num_cores=NC, num_subcores=NS)

    emb = functools.partial(
        pl.kernel,
        out_type=jax.ShapeDtypeStruct((N, D), jnp.float32),
        mesh=mesh,
        scratch_types=[
            pltpu.VMEM((CHUNK,), jnp.int32),
            pltpu.VMEM((CHUNK,), jnp.int32),
            pltpu.VMEM((CHUNK, D), jnp.float32),
            pltpu.VMEM((2 * D,), jnp.float32),
            pltpu.SemaphoreType.DMA,
        ],
        compiler_params=pltpu.CompilerParams(use_tc_tiling_on_sc=False),
    )(functools.partial(_emb_body, per_w, n_chunks, D))

    out = emb(ids, tids, token_table, typ)
    return out.reshape(B, S, D)


# speedup vs baseline: 13.7470x; 1.1190x over previous
import functools

import jax
import jax.numpy as jnp
from jax import lax
from jax.experimental import pallas as pl
from jax.experimental.pallas import tpu as pltpu
from jax.experimental.pallas import tpu_sc as plsc

NC = 2
NS = 16
LANES = 16
CHUNK = 256
DEPTH = 4


def _emb_body(per_w, n_chunks, D,
              ids_hbm, tids_hbm, tok_hbm, typ_hbm, out_hbm,
              idx_v, tid_v, typ_v,
              rows0, rows1, rows2, rows3,
              g0, g1, g2, g3, s0, s1, s2, s3):
    rows = [rows0, rows1, rows2, rows3]
    gsem = [g0, g1, g2, g3]
    ssem = [s0, s1, s2, s3]
    wid = lax.axis_index("s") * NC + lax.axis_index("c")
    w_base = wid * per_w
    nd = D // LANES

    pltpu.sync_copy(ids_hbm.at[pl.ds(w_base, per_w)], idx_v)
    pltpu.sync_copy(tids_hbm.at[pl.ds(w_base, per_w)], tid_v)
    pltpu.sync_copy(typ_hbm, typ_v)
    row0 = [typ_v[pl.ds(d * LANES, LANES)] for d in range(nd)]
    diff = [typ_v[pl.ds(D + d * LANES, LANES)] - row0[d] for d in range(nd)]

    def gather(ci, b):
        return pltpu.async_copy(
            tok_hbm.at[idx_v.at[pl.ds(ci * CHUNK, CHUNK)]], rows[b], gsem[b])

    gather(0, 0)
    gather(1, 1)

    def super_body(si, carry):
        for p in range(DEPTH):
            ci = si * DEPTH + p
            b = p
            bn = (p + 2) % DEPTH

            @pl.when(ci >= 2)
            def _():
                pltpu.make_async_copy(rows[bn],
                                      out_hbm.at[pl.ds(0, CHUNK)],
                                      ssem[bn]).wait()

            @pl.when(ci + 2 < n_chunks)
            def _():
                gather(ci + 2, bn)

            pltpu.make_async_copy(
                tok_hbm.at[idx_v.at[pl.ds(ci * CHUNK, CHUNK)]],
                rows[b], gsem[b]).wait()

            @plsc.parallel_loop(0, CHUNK, step=LANES, unroll=2)
            def add_body(t0):
                tid16 = tid_v[pl.ds(ci * CHUNK + t0, LANES)].astype(
                    jnp.float32)
                for l in range(LANES):
                    tidf = tid16[l]
                    for d in range(nd):
                        sl = pl.ds(d * LANES, LANES)
                        rows[b][t0 + l, sl] = rows[b][t0 + l, sl] + (
                            row0[d] + tidf * diff[d])

            pltpu.async_copy(
                rows[b], out_hbm.at[pl.ds(w_base + ci * CHUNK, CHUNK)],
                ssem[b])
        return carry

    lax.fori_loop(0, n_chunks // DEPTH, super_body, 0)

    for b in ((n_chunks - 2) % DEPTH, (n_chunks - 1) % DEPTH):
        pltpu.make_async_copy(rows[b], out_hbm.at[pl.ds(0, CHUNK)],
                              ssem[b]).wait()


def kernel(input_ids, type_ids, token_table, type_table):
    B, S = input_ids.shape
    V, D = token_table.shape
    N = B * S
    NW = NC * NS
    per_w = N // NW
    n_chunks = per_w // CHUNK
    assert per_w * NW == N and n_chunks * CHUNK == per_w
    assert n_chunks % DEPTH == 0

    ids = input_ids.reshape(N).astype(jnp.int32)
    tids = type_ids.reshape(N).astype(jnp.int32)
    typ = type_table.reshape(2 * D)

    mesh = plsc.VectorSubcoreMesh(
        core_axis_name="c", subcore_axis_name="s",
        num_cores=NC, num_subcores=NS)

    emb = functools.partial(
        pl.kernel,
        out_type=jax.ShapeDtypeStruct((N, D), jnp.float32),
        mesh=mesh,
        scratch_types=[
            pltpu.VMEM((per_w,), jnp.int32),
            pltpu.VMEM((per_w,), jnp.int32),
            pltpu.VMEM((2 * D,), jnp.float32),
        ] + [pltpu.VMEM((CHUNK, D), jnp.float32)] * DEPTH
          + [pltpu.SemaphoreType.DMA] * (2 * DEPTH),
        compiler_params=pltpu.CompilerParams(use_tc_tiling_on_sc=False),
    )(functools.partial(_emb_body, per_w, n_chunks, D))

    out = emb(ids, tids, token_table, typ)
    return out.reshape(B, S, D)


# speedup vs baseline: 13.7726x; 1.0019x over previous
import functools

import jax
import jax.numpy as jnp
from jax import lax
from jax.experimental import pallas as pl
from jax.experimental.pallas import tpu as pltpu
from jax.experimental.pallas import tpu_sc as plsc

NC = 2
NS = 16
LANES = 16
DEPTH = 4


def _emb_body(rows_per_w, n_rows, S, D,
              ids_hbm, tids_hbm, tok_hbm, typ_hbm, out_hbm,
              idx_v, tid_v, typ_v,
              rows0, rows1, rows2, rows3,
              g0, g1, g2, g3, s0, s1, s2, s3):
    rows = [rows0, rows1, rows2, rows3]
    gsem = [g0, g1, g2, g3]
    ssem = [s0, s1, s2, s3]
    wid = lax.axis_index("s") * NC + lax.axis_index("c")
    w_lo = wid * rows_per_w
    nd = D // LANES
    n_full = (S // LANES) * LANES

    pltpu.sync_copy(ids_hbm.at[pl.ds(w_lo, rows_per_w)], idx_v)
    pltpu.sync_copy(tids_hbm.at[pl.ds(w_lo, rows_per_w)], tid_v)
    pltpu.sync_copy(typ_hbm, typ_v)
    row0 = [typ_v[pl.ds(d * LANES, LANES)] for d in range(nd)]
    diff = [typ_v[pl.ds(D + d * LANES, LANES)] - row0[d] for d in range(nd)]

    def gather(r, b):
        return pltpu.async_copy(tok_hbm.at[idx_v.at[r]], rows[b], gsem[b])

    gather(0, 0)
    gather(1, 1)

    def add_block(b, r, t0, tid16, lanes):
        for l in lanes:
            tidf = tid16[l]
            for d in range(nd):
                sl = pl.ds(d * LANES, LANES)
                rows[b][t0 + l, sl] = rows[b][t0 + l, sl] + (
                    row0[d] + tidf * diff[d])

    def super_body(si, carry):
        for p in range(DEPTH):
            r = si * DEPTH + p
            b = p
            bn = (p + 2) % DEPTH

            @pl.when(r >= 2)
            def _():
                pltpu.make_async_copy(rows[bn], out_hbm.at[0],
                                      ssem[bn]).wait()

            @pl.when(r + 2 < n_rows)
            def _():
                gather(r + 2, bn)

            pltpu.make_async_copy(tok_hbm.at[idx_v.at[r]], rows[b],
                                  gsem[b]).wait()

            @plsc.parallel_loop(0, n_full, step=LANES, unroll=2)
            def add_body(t0):
                tid16 = tid_v[r, pl.ds(t0, LANES)].astype(jnp.float32)
                add_block(b, r, t0, tid16, range(LANES))

            if n_full < S:
                t0 = S - LANES
                tid16 = tid_v[r, pl.ds(t0, LANES)].astype(jnp.float32)
                add_block(b, r, t0, tid16, range(n_full - t0, LANES))

            pltpu.async_copy(rows[b], out_hbm.at[w_lo + r], ssem[b])
        return carry

    lax.fori_loop(0, n_rows // DEPTH, super_body, 0)

    for b in ((n_rows - 2) % DEPTH, (n_rows - 1) % DEPTH):
        pltpu.make_async_copy(rows[b], out_hbm.at[0], ssem[b]).wait()


def kernel(input_ids, type_ids, token_table, type_table):
    B, S = input_ids.shape
    V, D = token_table.shape
    NW = NC * NS
    rows_per_w = B // NW
    n_rows = rows_per_w
    assert rows_per_w * NW == B and n_rows % DEPTH == 0

    ids = input_ids.astype(jnp.int32)
    tids = type_ids.astype(jnp.int32)
    typ = type_table.reshape(2 * D)

    mesh = plsc.VectorSubcoreMesh(
        core_axis_name="c", subcore_axis_name="s",
        num_cores=NC, num_subcores=NS)

    emb = functools.partial(
        pl.kernel,
        out_type=jax.ShapeDtypeStruct((B, S, D), jnp.float32),
        mesh=mesh,
        scratch_types=[
            pltpu.VMEM((rows_per_w, S), jnp.int32),
            pltpu.VMEM((rows_per_w, S), jnp.int32),
            pltpu.VMEM((2 * D,), jnp.float32),
        ] + [pltpu.VMEM((S, D), jnp.float32)] * DEPTH
          + [pltpu.SemaphoreType.DMA] * (2 * DEPTH),
        compiler_params=pltpu.CompilerParams(use_tc_tiling_on_sc=False),
    )(functools.partial(_emb_body, rows_per_w, n_rows, S, D))

    return emb(ids, tids, token_table, typ)
